# Initial kernel scaffold; baseline (speedup 1.0000x reference)
import jax
import jax.numpy as jnp
from jax.experimental import pallas as pl


def kernel(c, p, n, c_emb, ctx_emb):
    raise NotImplementedError("write your pallas kernel here")


# speedup vs baseline: 4.8183x; 4.8183x over previous
import functools

import jax
import jax.numpy as jnp
from jax import lax
from jax.experimental import pallas as pl
from jax.experimental.pallas import tpu as pltpu
from jax.experimental.pallas import tpu_sc as plsc

N_TYPES = 100000
EMB_DIM = 32
BATCH = 16384
N_NEG = 20

NC = 2
NS = 16
L = 16
NW = NC * NS
B_PER_W = BATCH // NW
CHUNK = 128
N_CHUNKS = B_PER_W // CHUNK
GROUPS = CHUNK // L
NROWS = CHUNK * N_NEG


def _sc_scores_kernel(c_hbm, p_hbm, n_hbm, cemb_hbm, ctxemb_hbm,
                      pos_out, neg_out,
                      c_idx_v, p_idx_v, n_idx_v,
                      c_rows, p_rows, n_rows,
                      pos_v, neg_v, sem):
    wid = lax.axis_index("s") * NC + lax.axis_index("c")

    for ch in range(N_CHUNKS):
        base = wid * B_PER_W + ch * CHUNK

        pltpu.sync_copy(c_hbm.at[pl.ds(base, CHUNK)], c_idx_v)
        pltpu.sync_copy(p_hbm.at[pl.ds(base, CHUNK)], p_idx_v)
        pltpu.sync_copy(n_hbm.at[pl.ds(base * N_NEG, NROWS)], n_idx_v)

        copies = [
            pltpu.async_copy(cemb_hbm.at[c_idx_v], c_rows, sem),
            pltpu.async_copy(ctxemb_hbm.at[p_idx_v], p_rows, sem),
        ]
        for j in range(NROWS // 128):
            copies.append(
                pltpu.async_copy(ctxemb_hbm.at[n_idx_v.at[pl.ds(j * 128, 128)]],
                                 n_rows.at[pl.ds(j * 128, 128)], sem))
        for cp in copies:
            cp.wait()

        for g in range(GROUPS):
            b_ids = lax.broadcasted_iota(jnp.int32, (L,), 0) + g * L
            nb = b_ids * N_NEG

            def d_body(d, accs, b_ids=b_ids, nb=nb):
                dd = jnp.full((L,), d, jnp.int32)
                cv = plsc.load_gather(c_rows, [b_ids, dd])
                pv = plsc.load_gather(p_rows, [b_ids, dd])
                new = [accs[0] + cv * pv]
                for k in range(N_NEG):
                    nv = plsc.load_gather(n_rows, [nb + k, dd])
                    new.append(accs[k + 1] + cv * nv)
                return tuple(new)

            accs = lax.fori_loop(
                0, EMB_DIM, d_body,
                tuple(jnp.zeros((L,), jnp.float32) for _ in range(N_NEG + 1)))

            pos_v[pl.ds(g * L, L)] = accs[0]
            for k in range(N_NEG):
                plsc.store_scatter(neg_v, [nb + k], accs[k + 1])

        pltpu.sync_copy(pos_v, pos_out.at[pl.ds(base, CHUNK)])
        pltpu.sync_copy(neg_v, neg_out.at[pl.ds(base * N_NEG, NROWS)])


_sc_scores = functools.partial(
    pl.kernel,
    out_type=[
        jax.ShapeDtypeStruct((BATCH,), jnp.float32),
        jax.ShapeDtypeStruct((BATCH * N_NEG,), jnp.float32),
    ],
    mesh=plsc.VectorSubcoreMesh(core_axis_name="c", subcore_axis_name="s"),
    compiler_params=pltpu.CompilerParams(needs_layout_passes=False,
                                         use_tc_tiling_on_sc=False),
    scratch_types=[
        pltpu.VMEM((CHUNK,), jnp.int32),
        pltpu.VMEM((CHUNK,), jnp.int32),
        pltpu.VMEM((NROWS,), jnp.int32),
        pltpu.VMEM((CHUNK, EMB_DIM), jnp.float32),
        pltpu.VMEM((CHUNK, EMB_DIM), jnp.float32),
        pltpu.VMEM((NROWS, EMB_DIM), jnp.float32),
        pltpu.VMEM((CHUNK,), jnp.float32),
        pltpu.VMEM((NROWS,), jnp.float32),
        pltpu.SemaphoreType.DMA,
    ],
)(_sc_scores_kernel)


def _loss_body(pos_ref, neg_ref, out_ref):
    ls_pos = jnp.sum(jax.nn.log_sigmoid(pos_ref[...]))
    ls_neg = jnp.sum(jax.nn.log_sigmoid(-neg_ref[...]))
    out_ref[0, 0] = -(ls_pos / BATCH + ls_neg / (BATCH * N_NEG))


_tc_loss = pl.pallas_call(
    _loss_body,
    out_shape=jax.ShapeDtypeStruct((1, 1), jnp.float32),
    in_specs=[
        pl.BlockSpec(memory_space=pltpu.VMEM),
        pl.BlockSpec(memory_space=pltpu.VMEM),
    ],
    out_specs=pl.BlockSpec(memory_space=pltpu.SMEM),
)


@jax.jit
def kernel(c, p, n, c_emb, ctx_emb):
    c = c.astype(jnp.int32)
    p = p.astype(jnp.int32)
    n = n.astype(jnp.int32).reshape(BATCH * N_NEG)
    pos, neg = _sc_scores(c, p, n, c_emb, ctx_emb)
    loss = _tc_loss(pos.reshape(BATCH // 128, 128),
                    neg.reshape(BATCH * N_NEG // 128, 128))
    return loss[0, 0]


# speedup vs baseline: 4.9422x; 1.0257x over previous
import functools

import jax
import jax.numpy as jnp
from jax import lax
from jax.experimental import pallas as pl
from jax.experimental.pallas import tpu as pltpu
from jax.experimental.pallas import tpu_sc as plsc

N_TYPES = 100000
EMB_DIM = 32
BATCH = 16384
N_NEG = 20

NC = 2
NS = 16
L = 16
NW = NC * NS
B_PER_W = BATCH // NW
CHUNK = 64
N_CHUNKS = B_PER_W // CHUNK
GROUPS = CHUNK // L
NROWS = CHUNK * N_NEG


def _sc_scores_kernel(c_hbm, p_hbm, n_hbm, cemb_hbm, ctxemb_hbm,
                      pos_out, neg_out, *scr):
    bufs = (scr[0:8], scr[8:16])
    sems = scr[16:18]
    wid = lax.axis_index("s") * NC + lax.axis_index("c")

    def issue(ch, par):
        c_idx, p_idx, n_idx, c_rows, p_rows, n_rows, _, _ = bufs[par]
        base = wid * B_PER_W + ch * CHUNK
        pltpu.sync_copy(c_hbm.at[pl.ds(base, CHUNK)], c_idx)
        pltpu.sync_copy(p_hbm.at[pl.ds(base, CHUNK)], p_idx)
        pltpu.sync_copy(n_hbm.at[pl.ds(base * N_NEG, NROWS)], n_idx)
        cps = [
            pltpu.async_copy(cemb_hbm.at[c_idx], c_rows, sems[par]),
            pltpu.async_copy(ctxemb_hbm.at[p_idx], p_rows, sems[par]),
        ]
        for j in range(NROWS // 128):
            cps.append(
                pltpu.async_copy(ctxemb_hbm.at[n_idx.at[pl.ds(j * 128, 128)]],
                                 n_rows.at[pl.ds(j * 128, 128)], sems[par]))
        return cps

    pending = [None, None]
    pending[0] = issue(0, 0)

    for ch in range(N_CHUNKS):
        par = ch & 1
        if ch + 1 < N_CHUNKS:
            pending[1 - par] = issue(ch + 1, 1 - par)

        for cp in pending[par]:
            cp.wait()

        _, _, _, c_rows, p_rows, n_rows, pos_v, neg_v = bufs[par]
        base = wid * B_PER_W + ch * CHUNK

        for g in range(GROUPS):
            b_ids = lax.broadcasted_iota(jnp.int32, (L,), 0) + g * L
            nb = b_ids * N_NEG

            def d_body(d, accs, c_rows=c_rows, p_rows=p_rows, n_rows=n_rows,
                       b_ids=b_ids, nb=nb):
                dd = jnp.full((L,), d, jnp.int32)
                cv = plsc.load_gather(c_rows, [b_ids, dd])
                pv = plsc.load_gather(p_rows, [b_ids, dd])
                new = [accs[0] + cv * pv]
                for k in range(N_NEG):
                    nv = plsc.load_gather(n_rows, [nb + k, dd])
                    new.append(accs[k + 1] + cv * nv)
                return tuple(new)

            accs = lax.fori_loop(
                0, EMB_DIM, d_body,
                tuple(jnp.zeros((L,), jnp.float32) for _ in range(N_NEG + 1)))

            pos_v[pl.ds(g * L, L)] = accs[0]
            for k in range(N_NEG):
                plsc.store_scatter(neg_v, [nb + k], accs[k + 1])

        pltpu.sync_copy(pos_v, pos_out.at[pl.ds(base, CHUNK)])
        pltpu.sync_copy(neg_v, neg_out.at[pl.ds(base * N_NEG, NROWS)])


def _parity_bufs():
    return [
        pltpu.VMEM((CHUNK,), jnp.int32),
        pltpu.VMEM((CHUNK,), jnp.int32),
        pltpu.VMEM((NROWS,), jnp.int32),
        pltpu.VMEM((CHUNK, EMB_DIM), jnp.float32),
        pltpu.VMEM((CHUNK, EMB_DIM), jnp.float32),
        pltpu.VMEM((NROWS, EMB_DIM), jnp.float32),
        pltpu.VMEM((CHUNK,), jnp.float32),
        pltpu.VMEM((NROWS,), jnp.float32),
    ]


_sc_scores = functools.partial(
    pl.kernel,
    out_type=[
        jax.ShapeDtypeStruct((BATCH,), jnp.float32),
        jax.ShapeDtypeStruct((BATCH * N_NEG,), jnp.float32),
    ],
    mesh=plsc.VectorSubcoreMesh(core_axis_name="c", subcore_axis_name="s"),
    compiler_params=pltpu.CompilerParams(needs_layout_passes=False,
                                         use_tc_tiling_on_sc=False),
    scratch_types=_parity_bufs() + _parity_bufs() + [
        pltpu.SemaphoreType.DMA,
        pltpu.SemaphoreType.DMA,
    ],
)(_sc_scores_kernel)


def _loss_body(pos_ref, neg_ref, out_ref):
    ls_pos = jnp.sum(jax.nn.log_sigmoid(pos_ref[...]))
    ls_neg = jnp.sum(jax.nn.log_sigmoid(-neg_ref[...]))
    out_ref[0, 0] = -(ls_pos / BATCH + ls_neg / (BATCH * N_NEG))


_tc_loss = pl.pallas_call(
    _loss_body,
    out_shape=jax.ShapeDtypeStruct((1, 1), jnp.float32),
    in_specs=[
        pl.BlockSpec(memory_space=pltpu.VMEM),
        pl.BlockSpec(memory_space=pltpu.VMEM),
    ],
    out_specs=pl.BlockSpec(memory_space=pltpu.SMEM),
)


@jax.jit
def kernel(c, p, n, c_emb, ctx_emb):
    c = c.astype(jnp.int32)
    p = p.astype(jnp.int32)
    n = n.astype(jnp.int32).reshape(BATCH * N_NEG)
    pos, neg = _sc_scores(c, p, n, c_emb, ctx_emb)
    loss = _tc_loss(pos.reshape(BATCH // 128, 128),
                    neg.reshape(BATCH * N_NEG // 128, 128))
    return loss[0, 0]
